# Initial kernel scaffold; baseline (speedup 1.0000x reference)
import functools

import jax
import jax.numpy as jnp
from jax import lax
from jax.experimental import pallas as pl
from jax.experimental.pallas import tpu as pltpu
from jax.experimental.pallas import tpu_sc as plsc

N = 10000
E = 160000
D_IN = 256
D_OUT = 256
ALPHA = 0.2

H = D_OUT // 2
NS = 16
EPT = E // NS
BLK = 80
NBLK = EPT // BLK
NPAD = 10240
RPT = NPAD // NS
OCH = 128
NOCH = RPT // OCH
L = 16


def _tc_body(x_ref, w_ref, a0_ref, a1_ref, xp0_ref, xp1_ref, s0_ref, s1_ref):
    xp = jnp.dot(x_ref[...], w_ref[...], preferred_element_type=jnp.float32)
    xp0_ref[...] = xp[:, :H]
    xp1_ref[...] = xp[:, H:]
    s0_ref[...] = jnp.sum(xp * a0_ref[...], axis=1, keepdims=True)
    s1_ref[...] = jnp.sum(xp * a1_ref[...], axis=1, keepdims=True)


def _tc_stage(X, W, a0, a1):
    RB = 400
    grid = (N // RB,)
    return pl.pallas_call(
        _tc_body,
        grid=grid,
        in_specs=[
            pl.BlockSpec((RB, D_IN), lambda i: (i, 0)),
            pl.BlockSpec((D_IN, D_OUT), lambda i: (0, 0)),
            pl.BlockSpec((1, D_OUT), lambda i: (0, 0)),
            pl.BlockSpec((1, D_OUT), lambda i: (0, 0)),
        ],
        out_specs=[
            pl.BlockSpec((RB, H), lambda i: (i, 0)),
            pl.BlockSpec((RB, H), lambda i: (i, 0)),
            pl.BlockSpec((RB, 1), lambda i: (i, 0)),
            pl.BlockSpec((RB, 1), lambda i: (i, 0)),
        ],
        out_shape=[
            jax.ShapeDtypeStruct((N, H), jnp.float32),
            jax.ShapeDtypeStruct((N, H), jnp.float32),
            jax.ShapeDtypeStruct((N, 1), jnp.float32),
            jax.ShapeDtypeStruct((N, 1), jnp.float32),
        ],
    )(X, W, a0, a1)


def _sc_body(xp0_hbm, xp1_hbm, s0_hbm, s1_hbm, si_hbm, di_hbm, out_hbm,
             s0_v, s1_v, src_v, dst_v, att_v, rows_v, ob_v, rsb_v,
             mm_v, mmrd_v, h_sh, rs_sh, mmn_sh, mmx_sh):
    cid = lax.axis_index("c")
    sid = lax.axis_index("s")

    zed = jnp.zeros((L,), jnp.float32)

    pltpu.sync_copy(s0_hbm, s0_v)
    pltpu.sync_copy(s1_hbm, s1_v)
    pltpu.sync_copy(si_hbm.at[sid], src_v)
    pltpu.sync_copy(di_hbm.at[sid], dst_v)

    @pl.loop(0, OCH)
    def _(i):
        for q in range(H // L):
            ob_v[i, pl.ds(q * L, L)] = zed

    @pl.loop(0, RPT, step=L)
    def _(i):
        rsb_v[pl.ds(i, L)] = zed

    @pl.loop(0, NOCH)
    def _(ch):
        pltpu.sync_copy(ob_v, h_sh.at[pl.ds(sid * RPT + ch * OCH, OCH)])

    pltpu.sync_copy(rsb_v, rs_sh.at[pl.ds(sid * RPT, RPT)])

    big = jnp.full((L,), jnp.float32(jnp.inf))

    def block_a(j, carry):
        mnv, mxv = carry
        for k in range(BLK // L):
            sl = pl.ds(k * L, L)
            sv = src_v[j, sl]
            dv = dst_v[j, sl]
            g = plsc.load_gather(s0_v, [dv]) + plsc.load_gather(s1_v, [sv])
            lr = jnp.where(g >= 0, g, jnp.float32(ALPHA) * g)
            att_v[j, sl] = lr
            mnv = jnp.minimum(mnv, lr)
            mxv = jnp.maximum(mxv, lr)
        return mnv, mxv

    mnv, mxv = lax.fori_loop(0, NBLK, block_a, (big, -big))

    mm_v[...] = mnv
    pltpu.sync_copy(mm_v, mmn_sh.at[sid])
    mm_v[...] = mxv
    pltpu.sync_copy(mm_v, mmx_sh.at[sid])
    plsc.subcore_barrier()
    pltpu.sync_copy(mmn_sh, mmrd_v)
    for i in range(NS):
        mnv = jnp.minimum(mnv, mmrd_v[i])
    pltpu.sync_copy(mmx_sh, mmrd_v)
    for i in range(NS):
        mxv = jnp.maximum(mxv, mmrd_v[i])
    mn_s = jnp.min(mnv)
    rng_s = jnp.max(mxv) - mn_s
    mn_bc = jnp.full((L,), mn_s)
    rng_bc = jnp.full((L,), rng_s)

    def block_bc(j, _):
        for k in range(BLK // L):
            sl = pl.ds(k * L, L)
            lr = att_v[j, sl]
            att_v[j, sl] = jnp.exp((lr - mn_bc) / rng_bc)
        pltpu.sync_copy(att_v.at[j], rs_sh.at[dst_v.at[j]], add=True)

        @pl.when(cid == 0)
        def _():
            pltpu.sync_copy(xp0_hbm.at[src_v.at[j]], rows_v)

        @pl.when(cid == 1)
        def _():
            pltpu.sync_copy(xp1_hbm.at[src_v.at[j]], rows_v)

        @pl.loop(0, BLK)
        def _(i):
            av = jnp.full((L,), att_v[j, i])
            for q in range(H // L):
                sl = pl.ds(q * L, L)
                rows_v[i, sl] = rows_v[i, sl] * av

        pltpu.sync_copy(rows_v, h_sh.at[dst_v.at[j]], add=True)
        return 0

    lax.fori_loop(0, NBLK, block_bc, 0)
    plsc.subcore_barrier()

    one = jnp.full((L,), jnp.float32(1.0))
    pltpu.sync_copy(rs_sh.at[pl.ds(sid * RPT, RPT)], rsb_v)

    @pl.loop(0, NOCH)
    def _(ch):
        r0 = sid * RPT + ch * OCH
        pltpu.sync_copy(h_sh.at[pl.ds(r0, OCH)], ob_v)

        @pl.loop(0, OCH)
        def _(i):
            rv = jnp.full((L,), rsb_v[ch * OCH + i])
            rv = jnp.where(rv == 0.0, one, rv)
            iv = one / rv
            for q in range(H // L):
                sl = pl.ds(q * L, L)
                ob_v[i, sl] = ob_v[i, sl] * iv

        pltpu.sync_copy(ob_v, out_hbm.at[pl.ds(r0, OCH), pl.ds(cid * H, H)])


def _sc_stage(xp0, xp1, s0, s1, si, di):
    mesh = plsc.VectorSubcoreMesh(core_axis_name="c", subcore_axis_name="s")
    f = pl.kernel(
        _sc_body,
        out_type=jax.ShapeDtypeStruct((NPAD, D_OUT), jnp.float32),
        mesh=mesh,
        scratch_types=[
            pltpu.VMEM((N,), jnp.float32),
            pltpu.VMEM((N,), jnp.float32),
            pltpu.VMEM((NBLK, BLK), jnp.int32),
            pltpu.VMEM((NBLK, BLK), jnp.int32),
            pltpu.VMEM((NBLK, BLK), jnp.float32),
            pltpu.VMEM((BLK, H), jnp.float32),
            pltpu.VMEM((OCH, H), jnp.float32),
            pltpu.VMEM((RPT,), jnp.float32),
            pltpu.VMEM((L,), jnp.float32),
            pltpu.VMEM((NS, L), jnp.float32),
            pltpu.VMEM_SHARED((NPAD, H), jnp.float32),
            pltpu.VMEM_SHARED((NPAD,), jnp.float32),
            pltpu.VMEM_SHARED((NS, L), jnp.float32),
            pltpu.VMEM_SHARED((NS, L), jnp.float32),
        ],
    )
    return f(xp0, xp1, s0, s1, si, di)


def kernel(X, edge_index, W, a0, a1):
    xp0, xp1, s0, s1 = _tc_stage(X, W, a0, a1)
    si = edge_index[0].reshape(NS, NBLK, BLK)
    di = edge_index[1].reshape(NS, NBLK, BLK)
    out = _sc_stage(xp0, xp1, s0.reshape(N), s1.reshape(N), si, di)
    return out[:N]


# speedup vs baseline: 4.5049x; 4.5049x over previous
import dataclasses
import functools

import jax
import jax.numpy as jnp
from jax import lax
from jax.experimental import pallas as pl
from jax.experimental.pallas import tpu as pltpu
from jax.experimental.pallas import tpu_sc as plsc

N = 10000
E = 160000
D_IN = 256
D_OUT = 256
ALPHA = 0.2

HQ = D_OUT // 4
NS = 16
EPT = E // NS
BLK = 80
NBLK = EPT // BLK
NPAD = 10240
RPT = NPAD // NS
OCH = 128
NOCH = RPT // OCH
L = 16


def _tc_body(x_ref, w_ref, a0_ref, a1_ref,
             xq0_ref, xq1_ref, xq2_ref, xq3_ref, s0_ref, s1_ref):
    xp = jnp.dot(x_ref[...], w_ref[...], preferred_element_type=jnp.float32)
    xq0_ref[...] = xp[:, 0 * HQ:1 * HQ]
    xq1_ref[...] = xp[:, 1 * HQ:2 * HQ]
    xq2_ref[...] = xp[:, 2 * HQ:3 * HQ]
    xq3_ref[...] = xp[:, 3 * HQ:4 * HQ]
    s0_ref[...] = jnp.sum(xp * a0_ref[...], axis=1, keepdims=True)
    s1_ref[...] = jnp.sum(xp * a1_ref[...], axis=1, keepdims=True)


def _tc_stage(X, W, a0, a1):
    RB = 400
    grid = (N // RB,)
    return pl.pallas_call(
        _tc_body,
        grid=grid,
        in_specs=[
            pl.BlockSpec((RB, D_IN), lambda i: (i, 0)),
            pl.BlockSpec((D_IN, D_OUT), lambda i: (0, 0)),
            pl.BlockSpec((1, D_OUT), lambda i: (0, 0)),
            pl.BlockSpec((1, D_OUT), lambda i: (0, 0)),
        ],
        out_specs=[
            pl.BlockSpec((RB, HQ), lambda i: (i, 0)),
            pl.BlockSpec((RB, HQ), lambda i: (i, 0)),
            pl.BlockSpec((RB, HQ), lambda i: (i, 0)),
            pl.BlockSpec((RB, HQ), lambda i: (i, 0)),
            pl.BlockSpec((RB, 1), lambda i: (i, 0)),
            pl.BlockSpec((RB, 1), lambda i: (i, 0)),
        ],
        out_shape=[
            jax.ShapeDtypeStruct((N, HQ), jnp.float32),
            jax.ShapeDtypeStruct((N, HQ), jnp.float32),
            jax.ShapeDtypeStruct((N, HQ), jnp.float32),
            jax.ShapeDtypeStruct((N, HQ), jnp.float32),
            jax.ShapeDtypeStruct((N, 1), jnp.float32),
            jax.ShapeDtypeStruct((N, 1), jnp.float32),
        ],
    )(X, W, a0, a1)


def _sc_body(xq0_hbm, xq1_hbm, xq2_hbm, xq3_hbm, s0_hbm, s1_hbm,
             si_hbm, di_hbm, out_hbm,
             s0_v, s1_v, src_v, dst_v, att_v, rows_v, ob_v, rsb_v,
             mm_v, mmrd_v, h_sh, rs_sh, mmn_sh, mmx_sh):
    cid = lax.axis_index("c")
    sid = lax.axis_index("s")

    zed = jnp.zeros((L,), jnp.float32)
    one = jnp.full((L,), jnp.float32(1.0))

    pltpu.sync_copy(s0_hbm, s0_v)
    pltpu.sync_copy(s1_hbm, s1_v)
    pltpu.sync_copy(si_hbm.at[sid], src_v)
    pltpu.sync_copy(di_hbm.at[sid], dst_v)

    def zero_accumulators(zero_rs):
        @pl.loop(0, OCH)
        def _(i):
            for q in range(HQ // L):
                ob_v[i, pl.ds(q * L, L)] = zed

        @pl.loop(0, NOCH)
        def _(ch):
            pltpu.sync_copy(ob_v, h_sh.at[pl.ds(sid * RPT + ch * OCH, OCH)])

        if zero_rs:
            @pl.loop(0, RPT, step=L)
            def _(i):
                rsb_v[pl.ds(i, L)] = zed

            pltpu.sync_copy(rsb_v, rs_sh.at[pl.ds(sid * RPT, RPT)])

    zero_accumulators(zero_rs=True)

    big = jnp.full((L,), jnp.float32(jnp.inf))

    def block_a(j, carry):
        mnv, mxv = carry
        for k in range(BLK // L):
            sl = pl.ds(k * L, L)
            sv = src_v[j, sl]
            dv = dst_v[j, sl]
            g = plsc.load_gather(s0_v, [dv]) + plsc.load_gather(s1_v, [sv])
            lr = jnp.where(g >= 0, g, jnp.float32(ALPHA) * g)
            att_v[j, sl] = lr
            mnv = jnp.minimum(mnv, lr)
            mxv = jnp.maximum(mxv, lr)
        return mnv, mxv

    mnv, mxv = lax.fori_loop(0, NBLK, block_a, (big, -big))

    mm_v[...] = mnv
    pltpu.sync_copy(mm_v, mmn_sh.at[sid])
    mm_v[...] = mxv
    pltpu.sync_copy(mm_v, mmx_sh.at[sid])
    plsc.subcore_barrier()
    pltpu.sync_copy(mmn_sh, mmrd_v)
    for i in range(NS):
        mnv = jnp.minimum(mnv, mmrd_v[i])
    pltpu.sync_copy(mmx_sh, mmrd_v)
    for i in range(NS):
        mxv = jnp.maximum(mxv, mmrd_v[i])
    mn_s = jnp.min(mnv)
    rng_s = jnp.max(mxv) - mn_s
    mn_bc = jnp.full((L,), mn_s)
    rng_bc = jnp.full((L,), rng_s)

    def scale_rows(j):
        @pl.loop(0, BLK, step=L)
        def _(i0):
            avv = att_v[j, pl.ds(i0, L)]
            for r in range(L):
                av = jnp.full((L,), avv[r])
                for q in range(HQ // L):
                    sl = pl.ds(q * L, L)
                    rows_v[i0 + r, sl] = rows_v[i0 + r, sl] * av

    def gather_quarter(j, qa_hbm, qb_hbm):
        @pl.when(cid == 0)
        def _():
            pltpu.sync_copy(qa_hbm.at[src_v.at[j]], rows_v)

        @pl.when(cid == 1)
        def _():
            pltpu.sync_copy(qb_hbm.at[src_v.at[j]], rows_v)

    def block_p0(j, _):
        for k in range(BLK // L):
            sl = pl.ds(k * L, L)
            lr = att_v[j, sl]
            att_v[j, sl] = jnp.exp((lr - mn_bc) / rng_bc)
        pltpu.sync_copy(att_v.at[j], rs_sh.at[dst_v.at[j]], add=True)
        gather_quarter(j, xq0_hbm, xq2_hbm)
        scale_rows(j)
        pltpu.sync_copy(rows_v, h_sh.at[dst_v.at[j]], add=True)
        return 0

    lax.fori_loop(0, NBLK, block_p0, 0)
    plsc.subcore_barrier()

    pltpu.sync_copy(rs_sh.at[pl.ds(sid * RPT, RPT)], rsb_v)

    def write_pass(p):
        @pl.loop(0, NOCH)
        def _(ch):
            r0 = sid * RPT + ch * OCH
            pltpu.sync_copy(h_sh.at[pl.ds(r0, OCH)], ob_v)

            @pl.loop(0, OCH, step=L)
            def _(i0):
                rvv = rsb_v[pl.ds(ch * OCH + i0, L)]
                rvv = jnp.where(rvv == 0.0, one, rvv)
                ivv = one / rvv
                for r in range(L):
                    iv = jnp.full((L,), ivv[r])
                    for q in range(HQ // L):
                        sl = pl.ds(q * L, L)
                        ob_v[i0 + r, sl] = ob_v[i0 + r, sl] * iv

            col0 = (2 * cid + p) * HQ
            pltpu.sync_copy(ob_v, out_hbm.at[pl.ds(r0, OCH), pl.ds(col0, HQ)])

    write_pass(0)
    plsc.subcore_barrier()

    zero_accumulators(zero_rs=False)
    plsc.subcore_barrier()

    def block_p1(j, _):
        gather_quarter(j, xq1_hbm, xq3_hbm)
        scale_rows(j)
        pltpu.sync_copy(rows_v, h_sh.at[dst_v.at[j]], add=True)
        return 0

    lax.fori_loop(0, NBLK, block_p1, 0)
    plsc.subcore_barrier()
    write_pass(1)


def _sc_stage(xq, s0, s1, si, di):
    mesh = plsc.VectorSubcoreMesh(core_axis_name="c", subcore_axis_name="s")
    cp = pltpu.CompilerParams(needs_layout_passes=False,
                              use_tc_tiling_on_sc=False)
    f = pl.kernel(
        _sc_body,
        compiler_params=cp,
        out_type=jax.ShapeDtypeStruct((NPAD, D_OUT), jnp.float32),
        mesh=mesh,
        scratch_types=[
            pltpu.VMEM((N,), jnp.float32),
            pltpu.VMEM((N,), jnp.float32),
            pltpu.VMEM((NBLK, BLK), jnp.int32),
            pltpu.VMEM((NBLK, BLK), jnp.int32),
            pltpu.VMEM((NBLK, BLK), jnp.float32),
            pltpu.VMEM((BLK, HQ), jnp.float32),
            pltpu.VMEM((OCH, HQ), jnp.float32),
            pltpu.VMEM((RPT,), jnp.float32),
            pltpu.VMEM((L,), jnp.float32),
            pltpu.VMEM((NS, L), jnp.float32),
            pltpu.VMEM_SHARED((NPAD, HQ), jnp.float32),
            pltpu.VMEM_SHARED((NPAD,), jnp.float32),
            pltpu.VMEM_SHARED((NS, L), jnp.float32),
            pltpu.VMEM_SHARED((NS, L), jnp.float32),
        ],
    )
    return f(*xq, s0, s1, si, di)


def kernel(X, edge_index, W, a0, a1):
    xq0, xq1, xq2, xq3, s0, s1 = _tc_stage(X, W, a0, a1)
    si = edge_index[0].reshape(NS, NBLK, BLK)
    di = edge_index[1].reshape(NS, NBLK, BLK)
    out = _sc_stage((xq0, xq1, xq2, xq3),
                    s0.reshape(N), s1.reshape(N), si, di)
    return out[:N]


# speedup vs baseline: 6.9848x; 1.5505x over previous
import dataclasses
import functools

import jax
import jax.numpy as jnp
from jax import lax
from jax.experimental import pallas as pl
from jax.experimental.pallas import tpu as pltpu
from jax.experimental.pallas import tpu_sc as plsc

N = 10000
E = 160000
D_IN = 256
D_OUT = 256
ALPHA = 0.2

HQ = D_OUT // 4
NS = 16
EPT = E // NS
BLK = 80
NBLK = EPT // BLK
NPAD = 10240
RPT = NPAD // NS
OCH = 128
NOCH = RPT // OCH
L = 16


def _tc_body(x_ref, w_ref, a0_ref, a1_ref,
             xq0_ref, xq1_ref, xq2_ref, xq3_ref, s0_ref, s1_ref):
    xp = jnp.dot(x_ref[...], w_ref[...], preferred_element_type=jnp.float32)
    xq0_ref[...] = xp[:, 0 * HQ:1 * HQ]
    xq1_ref[...] = xp[:, 1 * HQ:2 * HQ]
    xq2_ref[...] = xp[:, 2 * HQ:3 * HQ]
    xq3_ref[...] = xp[:, 3 * HQ:4 * HQ]
    s0_ref[...] = jnp.sum(xp * a0_ref[...], axis=1, keepdims=True)
    s1_ref[...] = jnp.sum(xp * a1_ref[...], axis=1, keepdims=True)


def _tc_stage(X, W, a0, a1):
    RB = 400
    grid = (N // RB,)
    return pl.pallas_call(
        _tc_body,
        grid=grid,
        in_specs=[
            pl.BlockSpec((RB, D_IN), lambda i: (i, 0)),
            pl.BlockSpec((D_IN, D_OUT), lambda i: (0, 0)),
            pl.BlockSpec((1, D_OUT), lambda i: (0, 0)),
            pl.BlockSpec((1, D_OUT), lambda i: (0, 0)),
        ],
        out_specs=[
            pl.BlockSpec((RB, HQ), lambda i: (i, 0)),
            pl.BlockSpec((RB, HQ), lambda i: (i, 0)),
            pl.BlockSpec((RB, HQ), lambda i: (i, 0)),
            pl.BlockSpec((RB, HQ), lambda i: (i, 0)),
            pl.BlockSpec((RB, 1), lambda i: (i, 0)),
            pl.BlockSpec((RB, 1), lambda i: (i, 0)),
        ],
        out_shape=[
            jax.ShapeDtypeStruct((N, HQ), jnp.float32),
            jax.ShapeDtypeStruct((N, HQ), jnp.float32),
            jax.ShapeDtypeStruct((N, HQ), jnp.float32),
            jax.ShapeDtypeStruct((N, HQ), jnp.float32),
            jax.ShapeDtypeStruct((N, 1), jnp.float32),
            jax.ShapeDtypeStruct((N, 1), jnp.float32),
        ],
    )(X, W, a0, a1)


def _sc_body(xq0_hbm, xq1_hbm, xq2_hbm, xq3_hbm, s0_hbm, s1_hbm,
             si_hbm, di_hbm, out_hbm,
             s0_v, s1_v, src_v, dst_v, att_v,
             rows0_v, rows1_v, rows2_v, rows3_v, ob_v, rsb_v,
             mm_v, mmrd_v, h_sh, rs_sh, mmn_sh, mmx_sh,
             gsem0, gsem1, gsem2, gsem3,
             ssem0, ssem1, ssem2, ssem3, rssem):
    cid = lax.axis_index("c")
    sid = lax.axis_index("s")

    zed = jnp.zeros((L,), jnp.float32)
    one = jnp.full((L,), jnp.float32(1.0))

    pltpu.sync_copy(s0_hbm, s0_v)
    pltpu.sync_copy(s1_hbm, s1_v)
    pltpu.sync_copy(si_hbm.at[sid], src_v)
    pltpu.sync_copy(di_hbm.at[sid], dst_v)

    def zero_accumulators(zero_rs):
        @pl.loop(0, OCH)
        def _(i):
            for q in range(HQ // L):
                ob_v[i, pl.ds(q * L, L)] = zed

        @pl.loop(0, NOCH)
        def _(ch):
            pltpu.sync_copy(ob_v, h_sh.at[pl.ds(sid * RPT + ch * OCH, OCH)])

        if zero_rs:
            @pl.loop(0, RPT, step=L)
            def _(i):
                rsb_v[pl.ds(i, L)] = zed

            pltpu.sync_copy(rsb_v, rs_sh.at[pl.ds(sid * RPT, RPT)])

    zero_accumulators(zero_rs=True)

    big = jnp.full((L,), jnp.float32(jnp.inf))

    def block_a(j, carry):
        mnv, mxv = carry
        for k in range(BLK // L):
            sl = pl.ds(k * L, L)
            sv = src_v[j, sl]
            dv = dst_v[j, sl]
            g = plsc.load_gather(s0_v, [dv]) + plsc.load_gather(s1_v, [sv])
            lr = jnp.where(g >= 0, g, jnp.float32(ALPHA) * g)
            att_v[j, sl] = lr
            mnv = jnp.minimum(mnv, lr)
            mxv = jnp.maximum(mxv, lr)
        return mnv, mxv

    mnv, mxv = lax.fori_loop(0, NBLK, block_a, (big, -big))

    mm_v[...] = mnv
    pltpu.sync_copy(mm_v, mmn_sh.at[sid])
    mm_v[...] = mxv
    pltpu.sync_copy(mm_v, mmx_sh.at[sid])
    plsc.subcore_barrier()
    pltpu.sync_copy(mmn_sh, mmrd_v)
    for i in range(NS):
        mnv = jnp.minimum(mnv, mmrd_v[i])
    pltpu.sync_copy(mmx_sh, mmrd_v)
    for i in range(NS):
        mxv = jnp.maximum(mxv, mmrd_v[i])
    mn_s = jnp.min(mnv)
    rng_s = jnp.max(mxv) - mn_s
    mn_bc = jnp.full((L,), mn_s)
    rng_bc = jnp.full((L,), rng_s)

    bufs = (rows0_v, rows1_v, rows2_v, rows3_v)
    gsems = (gsem0, gsem1, gsem2, gsem3)
    ssems = (ssem0, ssem1, ssem2, ssem3)

    def scale_rows(j, buf):
        @pl.loop(0, BLK, step=L)
        def _(i0):
            avv = att_v[j, pl.ds(i0, L)]
            for r in range(L):
                av = jnp.full((L,), avv[r])
                for q in range(HQ // L):
                    sl = pl.ds(q * L, L)
                    buf[i0 + r, sl] = buf[i0 + r, sl] * av

    def run_pass(qa_hbm, qb_hbm):
        def g_start(j, buf, sem):
            @pl.when(cid == 0)
            def _():
                pltpu.async_copy(qa_hbm.at[src_v.at[j]], buf, sem)

            @pl.when(cid == 1)
            def _():
                pltpu.async_copy(qb_hbm.at[src_v.at[j]], buf, sem)

        def g_wait(buf, sem):
            @pl.when(cid == 0)
            def _():
                pltpu.make_async_copy(qa_hbm.at[src_v.at[0]], buf, sem).wait()

            @pl.when(cid == 1)
            def _():
                pltpu.make_async_copy(qb_hbm.at[src_v.at[0]], buf, sem).wait()

        def s_start(j, buf, sem):
            pltpu.async_copy(buf, h_sh.at[dst_v.at[j]], sem, add=True)

        def s_wait(buf, sem):
            pltpu.make_async_copy(buf, h_sh.at[dst_v.at[0]], sem).wait()

        for b in range(4):
            g_start(b, bufs[b], gsems[b])

        def quad(i, _):
            for b in range(4):
                j = 4 * i + b
                g_wait(bufs[b], gsems[b])
                scale_rows(j, bufs[b])
                s_start(j, bufs[b], ssems[b])
                bp = (b + 3) % 4
                ok = (j >= 1) & (j + 3 < NBLK)

                @pl.when(ok)
                def _():
                    s_wait(bufs[bp], ssems[bp])
                    g_start(j + 3, bufs[bp], gsems[bp])
            return 0

        lax.fori_loop(0, NBLK // 4, quad, 0)
        bl = (NBLK - 1) % 4
        g_wait(bufs[bl], gsems[bl])
        scale_rows(NBLK - 1, bufs[bl])
        s_start(NBLK - 1, bufs[bl], ssems[bl])
        for b in range(4):
            s_wait(bufs[b], ssems[b])

    def block_b(j, _):
        for k in range(BLK // L):
            sl = pl.ds(k * L, L)
            lr = att_v[j, sl]
            att_v[j, sl] = jnp.exp((lr - mn_bc) / rng_bc)
        pltpu.async_copy(att_v.at[j], rs_sh.at[dst_v.at[j]], rssem, add=True)
        return 0

    lax.fori_loop(0, NBLK, block_b, 0)

    run_pass(xq0_hbm, xq2_hbm)

    def drain_rs(j, _):
        pltpu.make_async_copy(att_v.at[j], rs_sh.at[dst_v.at[j]], rssem).wait()
        return 0

    lax.fori_loop(0, NBLK, drain_rs, 0)
    plsc.subcore_barrier()

    pltpu.sync_copy(rs_sh.at[pl.ds(sid * RPT, RPT)], rsb_v)

    def write_pass(p):
        @pl.loop(0, NOCH)
        def _(ch):
            r0 = sid * RPT + ch * OCH
            pltpu.sync_copy(h_sh.at[pl.ds(r0, OCH)], ob_v)

            @pl.loop(0, OCH, step=L)
            def _(i0):
                rvv = rsb_v[pl.ds(ch * OCH + i0, L)]
                rvv = jnp.where(rvv == 0.0, one, rvv)
                ivv = one / rvv
                for r in range(L):
                    iv = jnp.full((L,), ivv[r])
                    for q in range(HQ // L):
                        sl = pl.ds(q * L, L)
                        ob_v[i0 + r, sl] = ob_v[i0 + r, sl] * iv

            col0 = (2 * cid + p) * HQ
            pltpu.sync_copy(ob_v, out_hbm.at[pl.ds(r0, OCH), pl.ds(col0, HQ)])

    write_pass(0)
    plsc.subcore_barrier()

    zero_accumulators(zero_rs=False)
    plsc.subcore_barrier()

    run_pass(xq1_hbm, xq3_hbm)
    plsc.subcore_barrier()
    write_pass(1)


def _sc_stage(xq, s0, s1, si, di):
    mesh = plsc.VectorSubcoreMesh(core_axis_name="c", subcore_axis_name="s")
    cp = pltpu.CompilerParams(needs_layout_passes=False,
                              use_tc_tiling_on_sc=False)
    f = pl.kernel(
        _sc_body,
        compiler_params=cp,
        out_type=jax.ShapeDtypeStruct((NPAD, D_OUT), jnp.float32),
        mesh=mesh,
        scratch_types=[
            pltpu.VMEM((N,), jnp.float32),
            pltpu.VMEM((N,), jnp.float32),
            pltpu.VMEM((NBLK, BLK), jnp.int32),
            pltpu.VMEM((NBLK, BLK), jnp.int32),
            pltpu.VMEM((NBLK, BLK), jnp.float32),
            pltpu.VMEM((BLK, HQ), jnp.float32),
            pltpu.VMEM((BLK, HQ), jnp.float32),
            pltpu.VMEM((BLK, HQ), jnp.float32),
            pltpu.VMEM((BLK, HQ), jnp.float32),
            pltpu.VMEM((OCH, HQ), jnp.float32),
            pltpu.VMEM((RPT,), jnp.float32),
            pltpu.VMEM((L,), jnp.float32),
            pltpu.VMEM((NS, L), jnp.float32),
            pltpu.VMEM_SHARED((NPAD, HQ), jnp.float32),
            pltpu.VMEM_SHARED((NPAD,), jnp.float32),
            pltpu.VMEM_SHARED((NS, L), jnp.float32),
            pltpu.VMEM_SHARED((NS, L), jnp.float32),
            pltpu.SemaphoreType.DMA,
            pltpu.SemaphoreType.DMA,
            pltpu.SemaphoreType.DMA,
            pltpu.SemaphoreType.DMA,
            pltpu.SemaphoreType.DMA,
            pltpu.SemaphoreType.DMA,
            pltpu.SemaphoreType.DMA,
            pltpu.SemaphoreType.DMA,
            pltpu.SemaphoreType.DMA,
        ],
    )
    return f(*xq, s0, s1, si, di)


def kernel(X, edge_index, W, a0, a1):
    xq0, xq1, xq2, xq3, s0, s1 = _tc_stage(X, W, a0, a1)
    si = edge_index[0].reshape(NS, NBLK, BLK)
    di = edge_index[1].reshape(NS, NBLK, BLK)
    out = _sc_stage((xq0, xq1, xq2, xq3),
                    s0.reshape(N), s1.reshape(N), si, di)
    return out[:N]


# speedup vs baseline: 12.8990x; 1.8467x over previous
import dataclasses
import functools

import numpy as np

import jax
import jax.numpy as jnp
from jax import lax
from jax.experimental import pallas as pl
from jax.experimental.pallas import tpu as pltpu
from jax.experimental.pallas import tpu_sc as plsc

N = 10000
E = 160000
D_IN = 256
D_OUT = 256
ALPHA = 0.2

HQ = D_OUT // 4
NS = 16
EPT = E // NS
BLK = 80
NBLK = EPT // BLK
RPT = 640
LAST_BASE = 15 * RPT
LAST_RPT = N - LAST_BASE
OCH = 128
LOCH = LAST_RPT // 5
NOCH = 5
L = 16

_PERM = np.array([32 * g + off for g in (0, 1)
                  for k in range(L) for off in (k, k + L)], np.int32)


def _tc_body(x_ref, w_ref, a0_ref, a1_ref, xp_ref, s0_ref, s1_ref):
    xp = jnp.dot(x_ref[...], w_ref[...], preferred_element_type=jnp.float32)
    xp_ref[...] = xp
    s0_ref[...] = jnp.sum(xp * a0_ref[...], axis=1, keepdims=True)
    s1_ref[...] = jnp.sum(xp * a1_ref[...], axis=1, keepdims=True)


def _tc_stage(X, W, a0, a1):
    RB = 400
    grid = (N // RB,)
    return pl.pallas_call(
        _tc_body,
        grid=grid,
        in_specs=[
            pl.BlockSpec((RB, D_IN), lambda i: (i, 0)),
            pl.BlockSpec((D_IN, D_OUT), lambda i: (0, 0)),
            pl.BlockSpec((1, D_OUT), lambda i: (0, 0)),
            pl.BlockSpec((1, D_OUT), lambda i: (0, 0)),
        ],
        out_specs=[
            pl.BlockSpec((RB, D_OUT), lambda i: (i, 0)),
            pl.BlockSpec((RB, 1), lambda i: (i, 0)),
            pl.BlockSpec((RB, 1), lambda i: (i, 0)),
        ],
        out_shape=[
            jax.ShapeDtypeStruct((N, D_OUT), jnp.float32),
            jax.ShapeDtypeStruct((N, 1), jnp.float32),
            jax.ShapeDtypeStruct((N, 1), jnp.float32),
        ],
    )(X, W, a0, a1)


def _sc_body(xq0_hbm, xq1_hbm, xq2_hbm, xq3_hbm, s0_hbm, s1_hbm,
             si_hbm, di_hbm, out_hbm,
             s0_v, s1_v, src_v, dst_v, att_v,
             g0_v, g1_v, g2_v, g3_v, c0_v, c1_v, c2_v, c3_v,
             ob0_v, rsb_v, mm_v, mmrd_v,
             h_sh, rs_sh, mmn_sh, mmx_sh,
             gsem0, gsem1, gsem2, gsem3,
             ssem0, ssem1, ssem2, ssem3, rssem):
    cid = lax.axis_index("c")
    sid = lax.axis_index("s")

    zed = jnp.zeros((L,), jnp.float32)
    one = jnp.full((L,), jnp.float32(1.0))

    pltpu.sync_copy(s0_hbm, s0_v)
    pltpu.sync_copy(s1_hbm, s1_v)
    pltpu.sync_copy(si_hbm.at[sid], src_v)
    pltpu.sync_copy(di_hbm.at[sid], dst_v)

    def zero_accumulators(zero_rs):
        @pl.loop(0, OCH)
        def _(i):
            for q in range(HQ // L):
                ob0_v[i, pl.ds(q * L, L)] = zed

        @pl.when(sid < 15)
        def _():
            @pl.loop(0, NOCH)
            def _(ch):
                pltpu.sync_copy(
                    ob0_v, h_sh.at[pl.ds(sid * RPT + ch * OCH, OCH)])

        @pl.when(sid == 15)
        def _():
            @pl.loop(0, NOCH)
            def _(ch):
                pltpu.sync_copy(
                    ob0_v.at[pl.ds(0, LOCH)],
                    h_sh.at[pl.ds(LAST_BASE + ch * LOCH, LOCH)])

        if zero_rs:
            @pl.loop(0, RPT, step=L)
            def _(i):
                rsb_v[pl.ds(i, L)] = zed

            @pl.when(sid < 15)
            def _():
                pltpu.sync_copy(rsb_v, rs_sh.at[pl.ds(sid * RPT, RPT)])

            @pl.when(sid == 15)
            def _():
                pltpu.sync_copy(rsb_v.at[pl.ds(0, LAST_RPT)],
                                rs_sh.at[pl.ds(LAST_BASE, LAST_RPT)])

    zero_accumulators(zero_rs=True)

    big = jnp.full((L,), jnp.float32(jnp.inf))

    def block_a(j, carry):
        mnv, mxv = carry
        for k in range(BLK // L):
            sl = pl.ds(k * L, L)
            sv = src_v[j, sl]
            dv = dst_v[j, sl]
            g = plsc.load_gather(s0_v, [dv]) + plsc.load_gather(s1_v, [sv])
            lr = jnp.where(g >= 0, g, jnp.float32(ALPHA) * g)
            att_v[j, sl] = lr
            mnv = jnp.minimum(mnv, lr)
            mxv = jnp.maximum(mxv, lr)
        return mnv, mxv

    mnv, mxv = lax.fori_loop(0, NBLK, block_a, (big, -big))

    mm_v[...] = mnv
    pltpu.sync_copy(mm_v, mmn_sh.at[sid])
    mm_v[...] = mxv
    pltpu.sync_copy(mm_v, mmx_sh.at[sid])
    plsc.subcore_barrier()
    pltpu.sync_copy(mmn_sh, mmrd_v)
    for i in range(NS):
        mnv = jnp.minimum(mnv, mmrd_v[i])
    pltpu.sync_copy(mmx_sh, mmrd_v)
    for i in range(NS):
        mxv = jnp.maximum(mxv, mmrd_v[i])
    mn_s = jnp.min(mnv)
    rng_s = jnp.max(mxv) - mn_s
    mn_bc = jnp.full((L,), mn_s)
    rng_bc = jnp.full((L,), rng_s)

    gbufs = (g0_v, g1_v, g2_v, g3_v)
    sbufs = (c0_v, c1_v, c2_v, c3_v)
    gsems = (gsem0, gsem1, gsem2, gsem3)
    ssems = (ssem0, ssem1, ssem2, ssem3)

    def scale_rows(j, gbuf, sbuf):
        @pl.loop(0, BLK, step=L)
        def _(i0):
            avv = att_v[j, pl.ds(i0, L)]
            for r0 in range(0, L, 4):
                outs = []
                for r in range(r0, r0 + 4):
                    av = jnp.full((L,), avv[r])
                    w0 = gbuf[i0 + r, pl.ds(0, 2 * L)]
                    w1 = gbuf[i0 + r, pl.ds(2 * L, 2 * L)]
                    a0, b0 = plsc.unpack(
                        w0, format=plsc.PackFormat.INTERLEAVED,
                        preferred_element_type=jnp.float32)
                    a1, b1 = plsc.unpack(
                        w1, format=plsc.PackFormat.INTERLEAVED,
                        preferred_element_type=jnp.float32)
                    for q, v in enumerate((a0, b0, a1, b1)):
                        outs.append((r, q, v * av))
                for r, q, v in outs:
                    sbuf[i0 + r, pl.ds(q * L, L)] = v

    def run_pass(qa_hbm, qb_hbm):
        def g_start(j, buf, sem):
            @pl.when(cid == 0)
            def _():
                pltpu.async_copy(qa_hbm.at[src_v.at[j]], buf, sem)

            @pl.when(cid == 1)
            def _():
                pltpu.async_copy(qb_hbm.at[src_v.at[j]], buf, sem)

        def g_wait(buf, sem):
            @pl.when(cid == 0)
            def _():
                pltpu.make_async_copy(qa_hbm.at[src_v.at[0]], buf, sem).wait()

            @pl.when(cid == 1)
            def _():
                pltpu.make_async_copy(qb_hbm.at[src_v.at[0]], buf, sem).wait()

        def s_start(j, buf, sem):
            pltpu.async_copy(buf, h_sh.at[dst_v.at[j]], sem, add=True)

        def s_wait(buf, sem):
            pltpu.make_async_copy(buf, h_sh.at[dst_v.at[0]], sem).wait()

        for b in range(4):
            g_start(b, gbufs[b], gsems[b])

        def quad(i, _):
            for b in range(4):
                j = 4 * i + b
                g_wait(gbufs[b], gsems[b])
                scale_rows(j, gbufs[b], sbufs[b])
                s_start(j, sbufs[b], ssems[b])
                bp = (b + 3) % 4
                ok = (j >= 1) & (j + 3 < NBLK)

                @pl.when(ok)
                def _():
                    s_wait(sbufs[bp], ssems[bp])
                    g_start(j + 3, gbufs[bp], gsems[bp])
            return 0

        lax.fori_loop(0, NBLK // 4, quad, 0)
        bl = (NBLK - 1) % 4
        g_wait(gbufs[bl], gsems[bl])
        scale_rows(NBLK - 1, gbufs[bl], sbufs[bl])
        s_start(NBLK - 1, sbufs[bl], ssems[bl])
        for b in range(4):
            s_wait(sbufs[b], ssems[b])

    def block_b(j, _):
        for k in range(BLK // L):
            sl = pl.ds(k * L, L)
            lr = att_v[j, sl]
            att_v[j, sl] = jnp.exp((lr - mn_bc) / rng_bc)
        pltpu.async_copy(att_v.at[j], rs_sh.at[dst_v.at[j]], rssem, add=True)
        return 0

    lax.fori_loop(0, NBLK, block_b, 0)

    run_pass(xq0_hbm, xq2_hbm)

    def drain_rs(j, _):
        pltpu.make_async_copy(att_v.at[j], rs_sh.at[dst_v.at[j]], rssem).wait()
        return 0

    lax.fori_loop(0, NBLK, drain_rs, 0)
    plsc.subcore_barrier()

    @pl.when(sid < 15)
    def _():
        pltpu.sync_copy(rs_sh.at[pl.ds(sid * RPT, RPT)], rsb_v)

    @pl.when(sid == 15)
    def _():
        pltpu.sync_copy(rs_sh.at[pl.ds(LAST_BASE, LAST_RPT)],
                        rsb_v.at[pl.ds(0, LAST_RPT)])


    def norm_chunk(ch, buf, rows):
        @pl.loop(0, rows, step=L)
        def _(i0):
            rvv = rsb_v[pl.ds(ch + i0, L)]
            rvv = jnp.where(rvv == 0.0, one, rvv)
            ivv = one / rvv
            for r0 in range(0, L, 4):
                outs = []
                for r in range(r0, r0 + 4):
                    iv = jnp.full((L,), ivv[r])
                    for q in range(HQ // L):
                        sl = pl.ds(q * L, L)
                        outs.append((r, sl, buf[i0 + r, sl] * iv))
                for r, sl, v in outs:
                    buf[i0 + r, sl] = v

    def write_pass(p):
        col0 = (2 * cid + p) * HQ

        @pl.when(sid < 15)
        def _():
            for ch in range(NOCH):
                r0 = sid * RPT + ch * OCH
                pltpu.sync_copy(h_sh.at[pl.ds(r0, OCH)], ob0_v)
                norm_chunk(ch * OCH, ob0_v, OCH)
                pltpu.sync_copy(
                    ob0_v,
                    out_hbm.at[pl.ds(r0, OCH), pl.ds(col0, HQ)])

        @pl.when(sid == 15)
        def _():
            for ch in range(NOCH):
                r0 = LAST_BASE + ch * LOCH
                pltpu.sync_copy(h_sh.at[pl.ds(r0, LOCH)],
                                ob0_v.at[pl.ds(0, LOCH)])
                norm_chunk(ch * LOCH, ob0_v, LOCH)
                pltpu.sync_copy(
                    ob0_v.at[pl.ds(0, LOCH)],
                    out_hbm.at[pl.ds(r0, LOCH), pl.ds(col0, HQ)])

    write_pass(0)
    plsc.subcore_barrier()

    zero_accumulators(zero_rs=False)
    plsc.subcore_barrier()
    run_pass(xq1_hbm, xq3_hbm)
    plsc.subcore_barrier()
    write_pass(1)


def _sc_stage(xq, s0, s1, si, di):
    mesh = plsc.VectorSubcoreMesh(core_axis_name="c", subcore_axis_name="s")
    cp = pltpu.CompilerParams(needs_layout_passes=False,
                              use_tc_tiling_on_sc=False)
    f = pl.kernel(
        _sc_body,
        compiler_params=cp,
        out_type=jax.ShapeDtypeStruct((N, D_OUT), jnp.float32),
        mesh=mesh,
        scratch_types=[
            pltpu.VMEM((N,), jnp.float32),
            pltpu.VMEM((N,), jnp.float32),
            pltpu.VMEM((NBLK, BLK), jnp.int32),
            pltpu.VMEM((NBLK, BLK), jnp.int32),
            pltpu.VMEM((NBLK, BLK), jnp.float32),
            pltpu.VMEM((BLK, HQ), jnp.bfloat16),
            pltpu.VMEM((BLK, HQ), jnp.bfloat16),
            pltpu.VMEM((BLK, HQ), jnp.bfloat16),
            pltpu.VMEM((BLK, HQ), jnp.bfloat16),
            pltpu.VMEM((BLK, HQ), jnp.float32),
            pltpu.VMEM((BLK, HQ), jnp.float32),
            pltpu.VMEM((BLK, HQ), jnp.float32),
            pltpu.VMEM((BLK, HQ), jnp.float32),
            pltpu.VMEM((OCH, HQ), jnp.float32),
            pltpu.VMEM((RPT,), jnp.float32),
            pltpu.VMEM((L,), jnp.float32),
            pltpu.VMEM((NS, L), jnp.float32),
            pltpu.VMEM_SHARED((N, HQ), jnp.float32),
            pltpu.VMEM_SHARED((N,), jnp.float32),
            pltpu.VMEM_SHARED((NS, L), jnp.float32),
            pltpu.VMEM_SHARED((NS, L), jnp.float32),
            pltpu.SemaphoreType.DMA,
            pltpu.SemaphoreType.DMA,
            pltpu.SemaphoreType.DMA,
            pltpu.SemaphoreType.DMA,
            pltpu.SemaphoreType.DMA,
            pltpu.SemaphoreType.DMA,
            pltpu.SemaphoreType.DMA,
            pltpu.SemaphoreType.DMA,
            pltpu.SemaphoreType.DMA,
        ],
    )
    return f(*xq, s0, s1, si, di)


def kernel(X, edge_index, W, a0, a1):
    xp, s0, s1 = _tc_stage(X, W, a0, a1)
    xqs = tuple(
        xp[:, q * HQ:(q + 1) * HQ].astype(jnp.bfloat16)[:, _PERM]
        for q in range(4))
    si = edge_index[0].reshape(NS, NBLK, BLK)
    di = edge_index[1].reshape(NS, NBLK, BLK)
    return _sc_stage(xqs, s0.reshape(N), s1.reshape(N), si, di)


# speedup vs baseline: 14.1011x; 1.0932x over previous
import dataclasses
import functools

import numpy as np

import jax
import jax.numpy as jnp
from jax import lax
from jax.experimental import pallas as pl
from jax.experimental.pallas import tpu as pltpu
from jax.experimental.pallas import tpu_sc as plsc

N = 10000
E = 160000
D_IN = 256
D_OUT = 256
ALPHA = 0.2

HQ = D_OUT // 4
NS = 16
EPT = E // NS
BLK = 80
NBLK = EPT // BLK
RPT = 640
LAST_BASE = 15 * RPT
LAST_RPT = N - LAST_BASE
OCH = 128
LOCH = LAST_RPT // 5
NOCH = 5
L = 16

def _tc_body(x_ref, w_ref, a0_ref, a1_ref, xp_ref, s0_ref, s1_ref):
    xp = jnp.dot(x_ref[...], w_ref[...], preferred_element_type=jnp.float32)
    xp_ref[...] = xp
    s0_ref[...] = jnp.sum(xp * a0_ref[...], axis=1, keepdims=True)
    s1_ref[...] = jnp.sum(xp * a1_ref[...], axis=1, keepdims=True)


def _tc_stage(X, W, a0, a1):
    RB = 400
    grid = (N // RB,)
    return pl.pallas_call(
        _tc_body,
        grid=grid,
        in_specs=[
            pl.BlockSpec((RB, D_IN), lambda i: (i, 0)),
            pl.BlockSpec((D_IN, D_OUT), lambda i: (0, 0)),
            pl.BlockSpec((1, D_OUT), lambda i: (0, 0)),
            pl.BlockSpec((1, D_OUT), lambda i: (0, 0)),
        ],
        out_specs=[
            pl.BlockSpec((RB, D_OUT), lambda i: (i, 0)),
            pl.BlockSpec((RB, 1), lambda i: (i, 0)),
            pl.BlockSpec((RB, 1), lambda i: (i, 0)),
        ],
        out_shape=[
            jax.ShapeDtypeStruct((N, D_OUT), jnp.float32),
            jax.ShapeDtypeStruct((N, 1), jnp.float32),
            jax.ShapeDtypeStruct((N, 1), jnp.float32),
        ],
    )(X, W, a0, a1)


def _sc_body(xq0_hbm, xq1_hbm, xq2_hbm, xq3_hbm, s0_hbm, s1_hbm,
             si_hbm, di_hbm, out_hbm,
             s0_v, s1_v, src_v, dst_v, att_v,
             g0_v, g1_v, g2_v, g3_v,
             ob0_v, rsb_v, mm_v, mmrd_v,
             h_sh, rs_sh, mmn_sh, mmx_sh,
             gsem0, gsem1, gsem2, gsem3,
             ssem0, ssem1, ssem2, ssem3, rssem):
    cid = lax.axis_index("c")
    sid = lax.axis_index("s")

    zed = jnp.zeros((L,), jnp.float32)
    one = jnp.full((L,), jnp.float32(1.0))

    pltpu.sync_copy(s0_hbm, s0_v)
    pltpu.sync_copy(s1_hbm, s1_v)
    pltpu.sync_copy(si_hbm.at[sid], src_v)
    pltpu.sync_copy(di_hbm.at[sid], dst_v)

    def zero_accumulators(zero_rs):
        @pl.loop(0, OCH)
        def _(i):
            for q in range(HQ // L):
                ob0_v[i, pl.ds(q * L, L)] = zed

        @pl.when(sid < 15)
        def _():
            @pl.loop(0, NOCH)
            def _(ch):
                pltpu.sync_copy(
                    ob0_v, h_sh.at[pl.ds(sid * RPT + ch * OCH, OCH)])

        @pl.when(sid == 15)
        def _():
            @pl.loop(0, NOCH)
            def _(ch):
                pltpu.sync_copy(
                    ob0_v.at[pl.ds(0, LOCH)],
                    h_sh.at[pl.ds(LAST_BASE + ch * LOCH, LOCH)])

        if zero_rs:
            @pl.loop(0, RPT, step=L)
            def _(i):
                rsb_v[pl.ds(i, L)] = zed

            @pl.when(sid < 15)
            def _():
                pltpu.sync_copy(rsb_v, rs_sh.at[pl.ds(sid * RPT, RPT)])

            @pl.when(sid == 15)
            def _():
                pltpu.sync_copy(rsb_v.at[pl.ds(0, LAST_RPT)],
                                rs_sh.at[pl.ds(LAST_BASE, LAST_RPT)])

    with jax.named_scope("zero0"):
        zero_accumulators(zero_rs=True)

    big = jnp.full((L,), jnp.float32(jnp.inf))

    def block_a(j, carry):
        mnv, mxv = carry
        for k in range(BLK // L):
            sl = pl.ds(k * L, L)
            sv = src_v[j, sl]
            dv = dst_v[j, sl]
            g = plsc.load_gather(s0_v, [dv]) + plsc.load_gather(s1_v, [sv])
            lr = jnp.where(g >= 0, g, jnp.float32(ALPHA) * g)
            att_v[j, sl] = lr
            mnv = jnp.minimum(mnv, lr)
            mxv = jnp.maximum(mxv, lr)
        return mnv, mxv

    with jax.named_scope("phaseA"):
        mnv, mxv = lax.fori_loop(0, NBLK, block_a, (big, -big))

    mm_v[...] = mnv
    pltpu.sync_copy(mm_v, mmn_sh.at[sid])
    mm_v[...] = mxv
    pltpu.sync_copy(mm_v, mmx_sh.at[sid])
    plsc.subcore_barrier()
    pltpu.sync_copy(mmn_sh, mmrd_v)
    for i in range(NS):
        mnv = jnp.minimum(mnv, mmrd_v[i])
    pltpu.sync_copy(mmx_sh, mmrd_v)
    for i in range(NS):
        mxv = jnp.maximum(mxv, mmrd_v[i])
    mn_s = jnp.min(mnv)
    rng_s = jnp.max(mxv) - mn_s
    mn_bc = jnp.full((L,), mn_s)
    rng_bc = jnp.full((L,), rng_s)

    gbufs = (g0_v, g1_v, g2_v, g3_v)
    gsems = (gsem0, gsem1, gsem2, gsem3)
    ssems = (ssem0, ssem1, ssem2, ssem3)

    def scale_rows(j, buf):
        @pl.loop(0, BLK, step=L)
        def _(i0):
            avv = att_v[j, pl.ds(i0, L)]
            for r0 in range(0, L, 4):
                outs = []
                for r in range(r0, r0 + 4):
                    av = jnp.full((L,), avv[r])
                    for q in range(HQ // L):
                        sl = pl.ds(q * L, L)
                        outs.append((r, sl, buf[i0 + r, sl] * av))
                for r, sl, v in outs:
                    buf[i0 + r, sl] = v

    def run_pass(qa_hbm, qb_hbm):
        def g_start(j, buf, sem):
            @pl.when(cid == 0)
            def _():
                pltpu.async_copy(qa_hbm.at[src_v.at[j]], buf, sem)

            @pl.when(cid == 1)
            def _():
                pltpu.async_copy(qb_hbm.at[src_v.at[j]], buf, sem)

        def g_wait(buf, sem):
            @pl.when(cid == 0)
            def _():
                pltpu.make_async_copy(qa_hbm.at[src_v.at[0]], buf, sem).wait()

            @pl.when(cid == 1)
            def _():
                pltpu.make_async_copy(qb_hbm.at[src_v.at[0]], buf, sem).wait()

        def s_start(j, buf, sem):
            pltpu.async_copy(buf, h_sh.at[dst_v.at[j]], sem, add=True)

        def s_wait(buf, sem):
            pltpu.make_async_copy(buf, h_sh.at[dst_v.at[0]], sem).wait()

        for b in range(4):
            g_start(b, gbufs[b], gsems[b])

        def quad(i, _):
            for b in range(4):
                j = 4 * i + b
                g_wait(gbufs[b], gsems[b])
                scale_rows(j, gbufs[b])
                s_start(j, gbufs[b], ssems[b])
                bp = (b + 3) % 4
                ok = (j >= 1) & (j + 3 < NBLK)

                @pl.when(ok)
                def _():
                    s_wait(gbufs[bp], ssems[bp])
                    g_start(j + 3, gbufs[bp], gsems[bp])
            return 0

        lax.fori_loop(0, NBLK // 4, quad, 0)
        bl = (NBLK - 1) % 4
        g_wait(gbufs[bl], gsems[bl])
        scale_rows(NBLK - 1, gbufs[bl])
        s_start(NBLK - 1, gbufs[bl], ssems[bl])
        for b in range(4):
            s_wait(gbufs[b], ssems[b])

    def block_b(j, _):
        for k in range(BLK // L):
            sl = pl.ds(k * L, L)
            lr = att_v[j, sl]
            att_v[j, sl] = jnp.exp((lr - mn_bc) / rng_bc)
        pltpu.async_copy(att_v.at[j], rs_sh.at[dst_v.at[j]], rssem, add=True)
        return 0

    with jax.named_scope("phaseB"):
        lax.fori_loop(0, NBLK, block_b, 0)

    with jax.named_scope("pass0"):
        run_pass(xq0_hbm, xq2_hbm)

    def drain_rs(j, _):
        pltpu.make_async_copy(att_v.at[j], rs_sh.at[dst_v.at[j]], rssem).wait()
        return 0

    with jax.named_scope("drain_rs"):
        lax.fori_loop(0, NBLK, drain_rs, 0)
    plsc.subcore_barrier()

    @pl.when(sid < 15)
    def _():
        pltpu.sync_copy(rs_sh.at[pl.ds(sid * RPT, RPT)], rsb_v)

    @pl.when(sid == 15)
    def _():
        pltpu.sync_copy(rs_sh.at[pl.ds(LAST_BASE, LAST_RPT)],
                        rsb_v.at[pl.ds(0, LAST_RPT)])


    def norm_chunk(ch, buf, rows):
        @pl.loop(0, rows, step=L)
        def _(i0):
            rvv = rsb_v[pl.ds(ch + i0, L)]
            rvv = jnp.where(rvv == 0.0, one, rvv)
            ivv = one / rvv
            for r0 in range(0, L, 4):
                outs = []
                for r in range(r0, r0 + 4):
                    iv = jnp.full((L,), ivv[r])
                    for q in range(HQ // L):
                        sl = pl.ds(q * L, L)
                        outs.append((r, sl, buf[i0 + r, sl] * iv))
                for r, sl, v in outs:
                    buf[i0 + r, sl] = v

    def write_pass(p):
        col0 = (2 * cid + p) * HQ

        @pl.when(sid < 15)
        def _():
            for ch in range(NOCH):
                r0 = sid * RPT + ch * OCH
                pltpu.sync_copy(h_sh.at[pl.ds(r0, OCH)], ob0_v)
                norm_chunk(ch * OCH, ob0_v, OCH)
                pltpu.sync_copy(
                    ob0_v,
                    out_hbm.at[pl.ds(r0, OCH), pl.ds(col0, HQ)])

        @pl.when(sid == 15)
        def _():
            for ch in range(NOCH):
                r0 = LAST_BASE + ch * LOCH
                pltpu.sync_copy(h_sh.at[pl.ds(r0, LOCH)],
                                ob0_v.at[pl.ds(0, LOCH)])
                norm_chunk(ch * LOCH, ob0_v, LOCH)
                pltpu.sync_copy(
                    ob0_v.at[pl.ds(0, LOCH)],
                    out_hbm.at[pl.ds(r0, LOCH), pl.ds(col0, HQ)])

    with jax.named_scope("writep0"):
        write_pass(0)
    plsc.subcore_barrier()

    with jax.named_scope("zero1"):
        zero_accumulators(zero_rs=False)
    plsc.subcore_barrier()
    with jax.named_scope("pass1"):
        run_pass(xq1_hbm, xq3_hbm)
    plsc.subcore_barrier()
    with jax.named_scope("writep1"):
        write_pass(1)


def _sc_stage(xq, s0, s1, si, di):
    mesh = plsc.VectorSubcoreMesh(core_axis_name="c", subcore_axis_name="s")
    cp = pltpu.CompilerParams(needs_layout_passes=False,
                              use_tc_tiling_on_sc=False)
    f = pl.kernel(
        _sc_body,
        compiler_params=cp,
        out_type=jax.ShapeDtypeStruct((N, D_OUT), jnp.float32),
        mesh=mesh,
        scratch_types=[
            pltpu.VMEM((N,), jnp.float32),
            pltpu.VMEM((N,), jnp.float32),
            pltpu.VMEM((NBLK, BLK), jnp.int32),
            pltpu.VMEM((NBLK, BLK), jnp.int32),
            pltpu.VMEM((NBLK, BLK), jnp.float32),
            pltpu.VMEM((BLK, HQ), jnp.float32),
            pltpu.VMEM((BLK, HQ), jnp.float32),
            pltpu.VMEM((BLK, HQ), jnp.float32),
            pltpu.VMEM((BLK, HQ), jnp.float32),
            pltpu.VMEM((OCH, HQ), jnp.float32),
            pltpu.VMEM((RPT,), jnp.float32),
            pltpu.VMEM((L,), jnp.float32),
            pltpu.VMEM((NS, L), jnp.float32),
            pltpu.VMEM_SHARED((N, HQ), jnp.float32),
            pltpu.VMEM_SHARED((N,), jnp.float32),
            pltpu.VMEM_SHARED((NS, L), jnp.float32),
            pltpu.VMEM_SHARED((NS, L), jnp.float32),
            pltpu.SemaphoreType.DMA,
            pltpu.SemaphoreType.DMA,
            pltpu.SemaphoreType.DMA,
            pltpu.SemaphoreType.DMA,
            pltpu.SemaphoreType.DMA,
            pltpu.SemaphoreType.DMA,
            pltpu.SemaphoreType.DMA,
            pltpu.SemaphoreType.DMA,
            pltpu.SemaphoreType.DMA,
        ],
    )
    return f(*xq, s0, s1, si, di)


def kernel(X, edge_index, W, a0, a1):
    xp, s0, s1 = _tc_stage(X, W, a0, a1)
    xqs = tuple(xp[:, q * HQ:(q + 1) * HQ] for q in range(4))
    si = edge_index[0].reshape(NS, NBLK, BLK)
    di = edge_index[1].reshape(NS, NBLK, BLK)
    return _sc_stage(xqs, s0.reshape(N), s1.reshape(N), si, di)


# speedup vs baseline: 15.3717x; 1.0901x over previous
import dataclasses
import functools

import numpy as np

import jax
import jax.numpy as jnp
from jax import lax
from jax.experimental import pallas as pl
from jax.experimental.pallas import tpu as pltpu
from jax.experimental.pallas import tpu_sc as plsc

N = 10000
E = 160000
D_IN = 256
D_OUT = 256
ALPHA = 0.2

HQ = D_OUT // 4
NS = 16
EPT = E // NS
BLK = 80
NBLK = EPT // BLK
RPT = 640
LAST_BASE = 15 * RPT
LAST_RPT = N - LAST_BASE
OCH = 128
LOCH = LAST_RPT // 5
NOCH = 5
L = 16

def _tc_body(x_ref, w_ref, a0_ref, a1_ref,
             xq0_ref, xq1_ref, xq2_ref, xq3_ref, s0_ref, s1_ref):
    xp = jnp.dot(x_ref[...], w_ref[...], preferred_element_type=jnp.float32)
    xq0_ref[...] = xp[:, 0 * HQ:1 * HQ]
    xq1_ref[...] = xp[:, 1 * HQ:2 * HQ]
    xq2_ref[...] = xp[:, 2 * HQ:3 * HQ]
    xq3_ref[...] = xp[:, 3 * HQ:4 * HQ]
    s0_ref[...] = jnp.sum(xp * a0_ref[...], axis=1, keepdims=True)
    s1_ref[...] = jnp.sum(xp * a1_ref[...], axis=1, keepdims=True)


def _tc_stage(X, W, a0, a1):
    RB = 1000
    grid = (N // RB,)
    return pl.pallas_call(
        _tc_body,
        grid=grid,
        in_specs=[
            pl.BlockSpec((RB, D_IN), lambda i: (i, 0)),
            pl.BlockSpec((D_IN, D_OUT), lambda i: (0, 0)),
            pl.BlockSpec((1, D_OUT), lambda i: (0, 0)),
            pl.BlockSpec((1, D_OUT), lambda i: (0, 0)),
        ],
        out_specs=[
            pl.BlockSpec((RB, HQ), lambda i: (i, 0)),
            pl.BlockSpec((RB, HQ), lambda i: (i, 0)),
            pl.BlockSpec((RB, HQ), lambda i: (i, 0)),
            pl.BlockSpec((RB, HQ), lambda i: (i, 0)),
            pl.BlockSpec((RB, 1), lambda i: (i, 0)),
            pl.BlockSpec((RB, 1), lambda i: (i, 0)),
        ],
        out_shape=[
            jax.ShapeDtypeStruct((N, HQ), jnp.float32),
            jax.ShapeDtypeStruct((N, HQ), jnp.float32),
            jax.ShapeDtypeStruct((N, HQ), jnp.float32),
            jax.ShapeDtypeStruct((N, HQ), jnp.float32),
            jax.ShapeDtypeStruct((N, 1), jnp.float32),
            jax.ShapeDtypeStruct((N, 1), jnp.float32),
        ],
    )(X, W, a0, a1)


def _sc_body(xq0_hbm, xq1_hbm, xq2_hbm, xq3_hbm, s0_hbm, s1_hbm,
             si_hbm, di_hbm, out_hbm,
             s0_v, s1_v, src_v, dst_v, att_v,
             g0_v, g1_v, g2_v, g3_v,
             ob0_v, rsb_v, mm_v, mmrd_v,
             h_sh, rs_sh, mmn_sh, mmx_sh,
             gsem0, gsem1, gsem2, gsem3,
             ssem0, ssem1, ssem2, ssem3, rssem):
    cid = lax.axis_index("c")
    sid = lax.axis_index("s")

    zed = jnp.zeros((L,), jnp.float32)
    one = jnp.full((L,), jnp.float32(1.0))

    pltpu.sync_copy(s0_hbm, s0_v)
    pltpu.sync_copy(s1_hbm, s1_v)
    pltpu.sync_copy(si_hbm.at[sid], src_v)
    pltpu.sync_copy(di_hbm.at[sid], dst_v)

    def zero_accumulators(zero_rs):
        @pl.loop(0, OCH)
        def _(i):
            for q in range(HQ // L):
                ob0_v[i, pl.ds(q * L, L)] = zed

        @pl.when(sid < 15)
        def _():
            @pl.loop(0, NOCH)
            def _(ch):
                pltpu.sync_copy(
                    ob0_v, h_sh.at[pl.ds(sid * RPT + ch * OCH, OCH)])

        @pl.when(sid == 15)
        def _():
            @pl.loop(0, NOCH)
            def _(ch):
                pltpu.sync_copy(
                    ob0_v.at[pl.ds(0, LOCH)],
                    h_sh.at[pl.ds(LAST_BASE + ch * LOCH, LOCH)])

        if zero_rs:
            @pl.loop(0, RPT, step=L)
            def _(i):
                rsb_v[pl.ds(i, L)] = zed

            @pl.when(sid < 15)
            def _():
                pltpu.sync_copy(rsb_v, rs_sh.at[pl.ds(sid * RPT, RPT)])

            @pl.when(sid == 15)
            def _():
                pltpu.sync_copy(rsb_v.at[pl.ds(0, LAST_RPT)],
                                rs_sh.at[pl.ds(LAST_BASE, LAST_RPT)])

    with jax.named_scope("zero0"):
        zero_accumulators(zero_rs=True)

    big = jnp.full((L,), jnp.float32(jnp.inf))

    def block_a(j, carry):
        mnv, mxv = carry
        for k in range(BLK // L):
            sl = pl.ds(k * L, L)
            sv = src_v[j, sl]
            dv = dst_v[j, sl]
            g = plsc.load_gather(s0_v, [dv]) + plsc.load_gather(s1_v, [sv])
            lr = jnp.where(g >= 0, g, jnp.float32(ALPHA) * g)
            att_v[j, sl] = lr
            mnv = jnp.minimum(mnv, lr)
            mxv = jnp.maximum(mxv, lr)
        return mnv, mxv

    with jax.named_scope("phaseA"):
        mnv, mxv = lax.fori_loop(0, NBLK, block_a, (big, -big))

    mm_v[...] = mnv
    pltpu.sync_copy(mm_v, mmn_sh.at[sid])
    mm_v[...] = mxv
    pltpu.sync_copy(mm_v, mmx_sh.at[sid])
    plsc.subcore_barrier()
    pltpu.sync_copy(mmn_sh, mmrd_v)
    for i in range(NS):
        mnv = jnp.minimum(mnv, mmrd_v[i])
    pltpu.sync_copy(mmx_sh, mmrd_v)
    for i in range(NS):
        mxv = jnp.maximum(mxv, mmrd_v[i])
    mn_s = jnp.min(mnv)
    rng_s = jnp.max(mxv) - mn_s
    mn_bc = jnp.full((L,), mn_s)
    rng_bc = jnp.full((L,), rng_s)

    gbufs = (g0_v, g1_v, g2_v, g3_v)
    gsems = (gsem0, gsem1, gsem2, gsem3)
    ssems = (ssem0, ssem1, ssem2, ssem3)

    def scale_rows(j, buf):
        @pl.loop(0, BLK, step=L)
        def _(i0):
            avv = att_v[j, pl.ds(i0, L)]
            for r0 in range(0, L, 4):
                outs = []
                for r in range(r0, r0 + 4):
                    av = jnp.full((L,), avv[r])
                    for q in range(HQ // L):
                        sl = pl.ds(q * L, L)
                        outs.append((r, sl, buf[i0 + r, sl] * av))
                for r, sl, v in outs:
                    buf[i0 + r, sl] = v

    def run_pass(qa_hbm, qb_hbm):
        def g_start(j, buf, sem):
            @pl.when(cid == 0)
            def _():
                pltpu.async_copy(qa_hbm.at[src_v.at[j]], buf, sem)

            @pl.when(cid == 1)
            def _():
                pltpu.async_copy(qb_hbm.at[src_v.at[j]], buf, sem)

        def g_wait(buf, sem):
            @pl.when(cid == 0)
            def _():
                pltpu.make_async_copy(qa_hbm.at[src_v.at[0]], buf, sem).wait()

            @pl.when(cid == 1)
            def _():
                pltpu.make_async_copy(qb_hbm.at[src_v.at[0]], buf, sem).wait()

        def s_start(j, buf, sem):
            pltpu.async_copy(buf, h_sh.at[dst_v.at[j]], sem, add=True)

        def s_wait(buf, sem):
            pltpu.make_async_copy(buf, h_sh.at[dst_v.at[0]], sem).wait()

        for b in range(4):
            g_start(b, gbufs[b], gsems[b])

        def quad(i, _):
            for b in range(4):
                j = 4 * i + b
                g_wait(gbufs[b], gsems[b])
                scale_rows(j, gbufs[b])
                s_start(j, gbufs[b], ssems[b])
                bp = (b + 3) % 4
                ok = (j >= 1) & (j + 3 < NBLK)

                @pl.when(ok)
                def _():
                    s_wait(gbufs[bp], ssems[bp])
                    g_start(j + 3, gbufs[bp], gsems[bp])
            return 0

        lax.fori_loop(0, NBLK // 4, quad, 0)
        bl = (NBLK - 1) % 4
        g_wait(gbufs[bl], gsems[bl])
        scale_rows(NBLK - 1, gbufs[bl])
        s_start(NBLK - 1, gbufs[bl], ssems[bl])
        for b in range(4):
            s_wait(gbufs[b], ssems[b])

    def block_b(j, _):
        for k in range(BLK // L):
            sl = pl.ds(k * L, L)
            lr = att_v[j, sl]
            att_v[j, sl] = jnp.exp((lr - mn_bc) / rng_bc)
        pltpu.async_copy(att_v.at[j], rs_sh.at[dst_v.at[j]], rssem, add=True)
        return 0

    with jax.named_scope("phaseB"):
        lax.fori_loop(0, NBLK, block_b, 0)

    with jax.named_scope("pass0"):
        run_pass(xq0_hbm, xq2_hbm)

    def drain_rs(j, _):
        pltpu.make_async_copy(att_v.at[j], rs_sh.at[dst_v.at[j]], rssem).wait()
        return 0

    with jax.named_scope("drain_rs"):
        lax.fori_loop(0, NBLK, drain_rs, 0)
    plsc.subcore_barrier()

    @pl.when(sid < 15)
    def _():
        pltpu.sync_copy(rs_sh.at[pl.ds(sid * RPT, RPT)], rsb_v)

    @pl.when(sid == 15)
    def _():
        pltpu.sync_copy(rs_sh.at[pl.ds(LAST_BASE, LAST_RPT)],
                        rsb_v.at[pl.ds(0, LAST_RPT)])


    def norm_chunk(ch, buf, rows):
        @pl.loop(0, rows, step=L)
        def _(i0):
            rvv = rsb_v[pl.ds(ch + i0, L)]
            rvv = jnp.where(rvv == 0.0, one, rvv)
            ivv = one / rvv
            for r0 in range(0, L, 4):
                outs = []
                for r in range(r0, r0 + 4):
                    iv = jnp.full((L,), ivv[r])
                    for q in range(HQ // L):
                        sl = pl.ds(q * L, L)
                        outs.append((r, sl, buf[i0 + r, sl] * iv))
                for r, sl, v in outs:
                    buf[i0 + r, sl] = v

    def write_pass(p):
        col0 = (2 * cid + p) * HQ

        @pl.when(sid < 15)
        def _():
            for ch in range(NOCH):
                r0 = sid * RPT + ch * OCH
                pltpu.sync_copy(h_sh.at[pl.ds(r0, OCH)], ob0_v)
                norm_chunk(ch * OCH, ob0_v, OCH)
                pltpu.sync_copy(
                    ob0_v,
                    out_hbm.at[pl.ds(r0, OCH), pl.ds(col0, HQ)])

        @pl.when(sid == 15)
        def _():
            for ch in range(NOCH):
                r0 = LAST_BASE + ch * LOCH
                pltpu.sync_copy(h_sh.at[pl.ds(r0, LOCH)],
                                ob0_v.at[pl.ds(0, LOCH)])
                norm_chunk(ch * LOCH, ob0_v, LOCH)
                pltpu.sync_copy(
                    ob0_v.at[pl.ds(0, LOCH)],
                    out_hbm.at[pl.ds(r0, LOCH), pl.ds(col0, HQ)])

    with jax.named_scope("writep0"):
        write_pass(0)
    plsc.subcore_barrier()

    with jax.named_scope("zero1"):
        zero_accumulators(zero_rs=False)
    plsc.subcore_barrier()
    with jax.named_scope("pass1"):
        run_pass(xq1_hbm, xq3_hbm)
    plsc.subcore_barrier()
    with jax.named_scope("writep1"):
        write_pass(1)


def _sc_stage(xq, s0, s1, si, di):
    mesh = plsc.VectorSubcoreMesh(core_axis_name="c", subcore_axis_name="s")
    cp = pltpu.CompilerParams(needs_layout_passes=False,
                              use_tc_tiling_on_sc=False)
    f = pl.kernel(
        _sc_body,
        compiler_params=cp,
        out_type=jax.ShapeDtypeStruct((N, D_OUT), jnp.float32),
        mesh=mesh,
        scratch_types=[
            pltpu.VMEM((N,), jnp.float32),
            pltpu.VMEM((N,), jnp.float32),
            pltpu.VMEM((NBLK, BLK), jnp.int32),
            pltpu.VMEM((NBLK, BLK), jnp.int32),
            pltpu.VMEM((NBLK, BLK), jnp.float32),
            pltpu.VMEM((BLK, HQ), jnp.float32),
            pltpu.VMEM((BLK, HQ), jnp.float32),
            pltpu.VMEM((BLK, HQ), jnp.float32),
            pltpu.VMEM((BLK, HQ), jnp.float32),
            pltpu.VMEM((OCH, HQ), jnp.float32),
            pltpu.VMEM((RPT,), jnp.float32),
            pltpu.VMEM((L,), jnp.float32),
            pltpu.VMEM((NS, L), jnp.float32),
            pltpu.VMEM_SHARED((N, HQ), jnp.float32),
            pltpu.VMEM_SHARED((N,), jnp.float32),
            pltpu.VMEM_SHARED((NS, L), jnp.float32),
            pltpu.VMEM_SHARED((NS, L), jnp.float32),
            pltpu.SemaphoreType.DMA,
            pltpu.SemaphoreType.DMA,
            pltpu.SemaphoreType.DMA,
            pltpu.SemaphoreType.DMA,
            pltpu.SemaphoreType.DMA,
            pltpu.SemaphoreType.DMA,
            pltpu.SemaphoreType.DMA,
            pltpu.SemaphoreType.DMA,
            pltpu.SemaphoreType.DMA,
        ],
    )
    return f(*xq, s0, s1, si, di)


def kernel(X, edge_index, W, a0, a1):
    xq0, xq1, xq2, xq3, s0, s1 = _tc_stage(X, W, a0, a1)
    si = edge_index[0].reshape(NS, NBLK, BLK)
    di = edge_index[1].reshape(NS, NBLK, BLK)
    return _sc_stage((xq0, xq1, xq2, xq3),
                     s0.reshape(N), s1.reshape(N), si, di)


# speedup vs baseline: 15.7484x; 1.0245x over previous
import jax
import jax.numpy as jnp
from jax import lax
from jax.experimental import pallas as pl
from jax.experimental.pallas import tpu as pltpu
from jax.experimental.pallas import tpu_sc as plsc

N = 10000
E = 160000
D_IN = 256
D_OUT = 256
ALPHA = 0.2

HQ = D_OUT // 4
NS = 16
EPT = E // NS
BLK = 80
NBLK = EPT // BLK
RPT = 640
LAST_BASE = 15 * RPT
LAST_RPT = N - LAST_BASE
OCH = 128
LOCH = LAST_RPT // 5
NOCH = 5
L = 16

def _tc_body(x_ref, w_ref, a0_ref, a1_ref,
             xq0_ref, xq1_ref, xq2_ref, xq3_ref, s0_ref, s1_ref):
    xp = jnp.dot(x_ref[...], w_ref[...], preferred_element_type=jnp.float32)
    xq0_ref[...] = xp[:, 0 * HQ:1 * HQ]
    xq1_ref[...] = xp[:, 1 * HQ:2 * HQ]
    xq2_ref[...] = xp[:, 2 * HQ:3 * HQ]
    xq3_ref[...] = xp[:, 3 * HQ:4 * HQ]
    s0_ref[...] = jnp.sum(xp * a0_ref[...], axis=1, keepdims=True)
    s1_ref[...] = jnp.sum(xp * a1_ref[...], axis=1, keepdims=True)


def _tc_stage(X, W, a0, a1):
    RB = 1000
    grid = (N // RB,)
    return pl.pallas_call(
        _tc_body,
        grid=grid,
        in_specs=[
            pl.BlockSpec((RB, D_IN), lambda i: (i, 0)),
            pl.BlockSpec((D_IN, D_OUT), lambda i: (0, 0)),
            pl.BlockSpec((1, D_OUT), lambda i: (0, 0)),
            pl.BlockSpec((1, D_OUT), lambda i: (0, 0)),
        ],
        out_specs=[
            pl.BlockSpec((RB, HQ), lambda i: (i, 0)),
            pl.BlockSpec((RB, HQ), lambda i: (i, 0)),
            pl.BlockSpec((RB, HQ), lambda i: (i, 0)),
            pl.BlockSpec((RB, HQ), lambda i: (i, 0)),
            pl.BlockSpec((RB, 1), lambda i: (i, 0)),
            pl.BlockSpec((RB, 1), lambda i: (i, 0)),
        ],
        out_shape=[
            jax.ShapeDtypeStruct((N, HQ), jnp.float32),
            jax.ShapeDtypeStruct((N, HQ), jnp.float32),
            jax.ShapeDtypeStruct((N, HQ), jnp.float32),
            jax.ShapeDtypeStruct((N, HQ), jnp.float32),
            jax.ShapeDtypeStruct((N, 1), jnp.float32),
            jax.ShapeDtypeStruct((N, 1), jnp.float32),
        ],
    )(X, W, a0, a1)


def _sc_body(xq0_hbm, xq1_hbm, xq2_hbm, xq3_hbm, s0_hbm, s1_hbm,
             ed_hbm, out_hbm,
             s0_v, s1_v, src_v, dst_v, att_v,
             g0_v, g1_v, g2_v, g3_v,
             ob0_v, rsb_v, mm_v, mmrd_v,
             h_sh, rs_sh, mmn_sh, mmx_sh,
             gsem0, gsem1, gsem2, gsem3,
             ssem0, ssem1, ssem2, ssem3, rssem):
    cid = lax.axis_index("c")
    sid = lax.axis_index("s")

    zed = jnp.zeros((L,), jnp.float32)
    one = jnp.full((L,), jnp.float32(1.0))

    pltpu.sync_copy(s0_hbm, s0_v)
    pltpu.sync_copy(s1_hbm, s1_v)
    pltpu.sync_copy(ed_hbm.at[0, sid], src_v)
    pltpu.sync_copy(ed_hbm.at[1, sid], dst_v)

    def zero_accumulators(zero_rs):
        @pl.loop(0, OCH)
        def _(i):
            for q in range(HQ // L):
                ob0_v[i, pl.ds(q * L, L)] = zed

        @pl.when(sid < 15)
        def _():
            @pl.loop(0, NOCH)
            def _(ch):
                pltpu.sync_copy(
                    ob0_v, h_sh.at[pl.ds(sid * RPT + ch * OCH, OCH)])

        @pl.when(sid == 15)
        def _():
            @pl.loop(0, NOCH)
            def _(ch):
                pltpu.sync_copy(
                    ob0_v.at[pl.ds(0, LOCH)],
                    h_sh.at[pl.ds(LAST_BASE + ch * LOCH, LOCH)])

        if zero_rs:
            @pl.loop(0, RPT, step=L)
            def _(i):
                rsb_v[pl.ds(i, L)] = zed

            @pl.when(sid < 15)
            def _():
                pltpu.sync_copy(rsb_v, rs_sh.at[pl.ds(sid * RPT, RPT)])

            @pl.when(sid == 15)
            def _():
                pltpu.sync_copy(rsb_v.at[pl.ds(0, LAST_RPT)],
                                rs_sh.at[pl.ds(LAST_BASE, LAST_RPT)])

    with jax.named_scope("zero0"):
        zero_accumulators(zero_rs=True)

    big = jnp.full((L,), jnp.float32(jnp.inf))

    def block_a(j, carry):
        mnv, mxv = carry
        for k in range(BLK // L):
            sl = pl.ds(k * L, L)
            sv = src_v[j, sl]
            dv = dst_v[j, sl]
            g = plsc.load_gather(s0_v, [dv]) + plsc.load_gather(s1_v, [sv])
            lr = jnp.where(g >= 0, g, jnp.float32(ALPHA) * g)
            att_v[j, sl] = lr
            mnv = jnp.minimum(mnv, lr)
            mxv = jnp.maximum(mxv, lr)
        return mnv, mxv

    with jax.named_scope("phaseA"):
        mnv, mxv = lax.fori_loop(0, NBLK, block_a, (big, -big))

    mm_v[...] = mnv
    pltpu.sync_copy(mm_v, mmn_sh.at[sid])
    mm_v[...] = mxv
    pltpu.sync_copy(mm_v, mmx_sh.at[sid])
    plsc.subcore_barrier()
    pltpu.sync_copy(mmn_sh, mmrd_v)
    for i in range(NS):
        mnv = jnp.minimum(mnv, mmrd_v[i])
    pltpu.sync_copy(mmx_sh, mmrd_v)
    for i in range(NS):
        mxv = jnp.maximum(mxv, mmrd_v[i])
    mn_s = jnp.min(mnv)
    rng_s = jnp.max(mxv) - mn_s
    mn_bc = jnp.full((L,), mn_s)
    rng_bc = jnp.full((L,), rng_s)

    gbufs = (g0_v, g1_v, g2_v, g3_v)
    gsems = (gsem0, gsem1, gsem2, gsem3)
    ssems = (ssem0, ssem1, ssem2, ssem3)

    def scale_rows(j, buf):
        @pl.loop(0, BLK, step=L)
        def _(i0):
            avv = att_v[j, pl.ds(i0, L)]
            for r0 in range(0, L, 4):
                outs = []
                for r in range(r0, r0 + 4):
                    av = jnp.full((L,), avv[r])
                    for q in range(HQ // L):
                        sl = pl.ds(q * L, L)
                        outs.append((r, sl, buf[i0 + r, sl] * av))
                for r, sl, v in outs:
                    buf[i0 + r, sl] = v

    def run_pass(qa_hbm, qb_hbm):
        def g_start(j, buf, sem):
            @pl.when(cid == 0)
            def _():
                pltpu.async_copy(qa_hbm.at[src_v.at[j]], buf, sem)

            @pl.when(cid == 1)
            def _():
                pltpu.async_copy(qb_hbm.at[src_v.at[j]], buf, sem)

        def g_wait(buf, sem):
            @pl.when(cid == 0)
            def _():
                pltpu.make_async_copy(qa_hbm.at[src_v.at[0]], buf, sem).wait()

            @pl.when(cid == 1)
            def _():
                pltpu.make_async_copy(qb_hbm.at[src_v.at[0]], buf, sem).wait()

        def s_start(j, buf, sem):
            pltpu.async_copy(buf, h_sh.at[dst_v.at[j]], sem, add=True)

        def s_wait(buf, sem):
            pltpu.make_async_copy(buf, h_sh.at[dst_v.at[0]], sem).wait()

        for b in range(4):
            g_start(b, gbufs[b], gsems[b])

        def quad(i, _):
            for b in range(4):
                j = 4 * i + b
                g_wait(gbufs[b], gsems[b])
                scale_rows(j, gbufs[b])
                s_start(j, gbufs[b], ssems[b])
                bp = (b + 3) % 4
                ok = (j >= 1) & (j + 3 < NBLK)

                @pl.when(ok)
                def _():
                    s_wait(gbufs[bp], ssems[bp])
                    g_start(j + 3, gbufs[bp], gsems[bp])
            return 0

        lax.fori_loop(0, NBLK // 4, quad, 0)
        bl = (NBLK - 1) % 4
        g_wait(gbufs[bl], gsems[bl])
        scale_rows(NBLK - 1, gbufs[bl])
        s_start(NBLK - 1, gbufs[bl], ssems[bl])
        for b in range(4):
            s_wait(gbufs[b], ssems[b])

    def block_b(j, _):
        for k in range(BLK // L):
            sl = pl.ds(k * L, L)
            lr = att_v[j, sl]
            att_v[j, sl] = jnp.exp((lr - mn_bc) / rng_bc)
        pltpu.async_copy(att_v.at[j], rs_sh.at[dst_v.at[j]], rssem, add=True)
        return 0

    with jax.named_scope("phaseB"):
        lax.fori_loop(0, NBLK, block_b, 0)

    with jax.named_scope("pass0"):
        run_pass(xq0_hbm, xq2_hbm)

    def drain_rs(j, _):
        pltpu.make_async_copy(att_v.at[j], rs_sh.at[dst_v.at[j]], rssem).wait()
        return 0

    with jax.named_scope("drain_rs"):
        lax.fori_loop(0, NBLK, drain_rs, 0)
    plsc.subcore_barrier()

    @pl.when(sid < 15)
    def _():
        pltpu.sync_copy(rs_sh.at[pl.ds(sid * RPT, RPT)], rsb_v)

    @pl.when(sid == 15)
    def _():
        pltpu.sync_copy(rs_sh.at[pl.ds(LAST_BASE, LAST_RPT)],
                        rsb_v.at[pl.ds(0, LAST_RPT)])


    def norm_chunk(ch, buf, rows):
        @pl.loop(0, rows, step=L)
        def _(i0):
            rvv = rsb_v[pl.ds(ch + i0, L)]
            rvv = jnp.where(rvv == 0.0, one, rvv)
            ivv = one / rvv
            for r0 in range(0, L, 4):
                outs = []
                for r in range(r0, r0 + 4):
                    iv = jnp.full((L,), ivv[r])
                    for q in range(HQ // L):
                        sl = pl.ds(q * L, L)
                        outs.append((r, sl, buf[i0 + r, sl] * iv))
                for r, sl, v in outs:
                    buf[i0 + r, sl] = v

    def write_pass(p):
        col0 = (2 * cid + p) * HQ

        @pl.when(sid < 15)
        def _():
            for ch in range(NOCH):
                r0 = sid * RPT + ch * OCH
                pltpu.sync_copy(h_sh.at[pl.ds(r0, OCH)], ob0_v)
                norm_chunk(ch * OCH, ob0_v, OCH)
                pltpu.sync_copy(
                    ob0_v,
                    out_hbm.at[pl.ds(r0, OCH), pl.ds(col0, HQ)])

        @pl.when(sid == 15)
        def _():
            for ch in range(NOCH):
                r0 = LAST_BASE + ch * LOCH
                pltpu.sync_copy(h_sh.at[pl.ds(r0, LOCH)],
                                ob0_v.at[pl.ds(0, LOCH)])
                norm_chunk(ch * LOCH, ob0_v, LOCH)
                pltpu.sync_copy(
                    ob0_v.at[pl.ds(0, LOCH)],
                    out_hbm.at[pl.ds(r0, LOCH), pl.ds(col0, HQ)])

    with jax.named_scope("writep0"):
        write_pass(0)
    plsc.subcore_barrier()

    with jax.named_scope("zero1"):
        zero_accumulators(zero_rs=False)
    plsc.subcore_barrier()
    with jax.named_scope("pass1"):
        run_pass(xq1_hbm, xq3_hbm)
    plsc.subcore_barrier()
    with jax.named_scope("writep1"):
        write_pass(1)


def _sc_stage(xq, s0, s1, ed):
    mesh = plsc.VectorSubcoreMesh(core_axis_name="c", subcore_axis_name="s")
    cp = pltpu.CompilerParams(needs_layout_passes=False,
                              use_tc_tiling_on_sc=False)
    f = pl.kernel(
        _sc_body,
        compiler_params=cp,
        out_type=jax.ShapeDtypeStruct((N, D_OUT), jnp.float32),
        mesh=mesh,
        scratch_types=[
            pltpu.VMEM((N,), jnp.float32),
            pltpu.VMEM((N,), jnp.float32),
            pltpu.VMEM((NBLK, BLK), jnp.int32),
            pltpu.VMEM((NBLK, BLK), jnp.int32),
            pltpu.VMEM((NBLK, BLK), jnp.float32),
            pltpu.VMEM((BLK, HQ), jnp.float32),
            pltpu.VMEM((BLK, HQ), jnp.float32),
            pltpu.VMEM((BLK, HQ), jnp.float32),
            pltpu.VMEM((BLK, HQ), jnp.float32),
            pltpu.VMEM((OCH, HQ), jnp.float32),
            pltpu.VMEM((RPT,), jnp.float32),
            pltpu.VMEM((L,), jnp.float32),
            pltpu.VMEM((NS, L), jnp.float32),
            pltpu.VMEM_SHARED((N, HQ), jnp.float32),
            pltpu.VMEM_SHARED((N,), jnp.float32),
            pltpu.VMEM_SHARED((NS, L), jnp.float32),
            pltpu.VMEM_SHARED((NS, L), jnp.float32),
            pltpu.SemaphoreType.DMA,
            pltpu.SemaphoreType.DMA,
            pltpu.SemaphoreType.DMA,
            pltpu.SemaphoreType.DMA,
            pltpu.SemaphoreType.DMA,
            pltpu.SemaphoreType.DMA,
            pltpu.SemaphoreType.DMA,
            pltpu.SemaphoreType.DMA,
            pltpu.SemaphoreType.DMA,
        ],
    )
    return f(*xq, s0, s1, ed)


def kernel(X, edge_index, W, a0, a1):
    xq0, xq1, xq2, xq3, s0, s1 = _tc_stage(X, W, a0, a1)
    ed = edge_index.reshape(2, NS, NBLK, BLK)
    return _sc_stage((xq0, xq1, xq2, xq3),
                     s0.reshape(N), s1.reshape(N), ed)
